# baseline (speedup 1.0000x reference)
import jax
import jax.numpy as jnp
from jax import lax
from jax.experimental import pallas as pl
from jax.experimental.pallas import tpu as pltpu
from jax.experimental.pallas import tpu_sc as plsc

NN = 10000
NE = 160000
NC = 2
NS = 16
NWORK = NC * NS
NEP = 163840
EPW = NEP // NWORK
CHUNK = 80
NCH = EPW // CHUNK
NNP = 10240
RPS = NNP // NS

_BR = 2000


def _sc_mesh():
    return plsc.VectorSubcoreMesh(core_axis_name="c", subcore_axis_name="s")


def _make_conv_kernel():
    dh = 128
    out_t = jax.ShapeDtypeStruct((NC, NNP, dh), jnp.float32)

    @pl.kernel(out_type=out_t, mesh=_sc_mesh(),
               scratch_types=[pltpu.VMEM((CHUNK,), jnp.int32),
                              pltpu.VMEM((CHUNK,), jnp.int32),
                              pltpu.VMEM((CHUNK, 16), jnp.float32),
                              pltpu.VMEM((CHUNK, 16), jnp.float32),
                              pltpu.VMEM((CHUNK, dh), jnp.float32),
                              pltpu.VMEM((CHUNK, dh), jnp.float32),
                              pltpu.VMEM((CHUNK,), jnp.int32),
                              pltpu.VMEM((CHUNK,), jnp.int32),
                              pltpu.VMEM_SHARED((NNP, dh), jnp.float32),
                              pltpu.SemaphoreType.DMA,
                              pltpu.SemaphoreType.DMA,
                              pltpu.SemaphoreType.DMA,
                              pltpu.SemaphoreType.DMA,
                              pltpu.SemaphoreType.DMA,
                              pltpu.SemaphoreType.DMA])
    def k(t_h, src_h, dst_h, w16_h, z_h, out_h,
          isrc0, idst0, wv0, wv1, rows0, rows1, isrc1, idst1, acc,
          sl0, sl1, sg0, sg1, ss0, ss1):
        c = lax.axis_index("c")
        s = lax.axis_index("s")
        wid = c * NS + s
        isrc = (isrc0, isrc1)
        idst = (idst0, idst1)
        wv = (wv0, wv1)
        rows = (rows0, rows1)
        sem_l = (sl0, sl1)
        sem_g = (sg0, sg1)
        sem_s = (ss0, ss1)

        def start_loads(b, kk):
            base = wid * EPW + kk * CHUNK
            pltpu.async_copy(src_h.at[pl.ds(base, CHUNK)], isrc[b], sem_l[b])
            pltpu.async_copy(dst_h.at[pl.ds(base, CHUNK)], idst[b], sem_l[b])
            pltpu.async_copy(w16_h.at[pl.ds(base, CHUNK)], wv[b], sem_l[b])

        def wait_loads(b):
            pltpu.make_async_copy(src_h.at[pl.ds(0, CHUNK)], isrc[b],
                                  sem_l[b]).wait()
            pltpu.make_async_copy(dst_h.at[pl.ds(0, CHUNK)], idst[b],
                                  sem_l[b]).wait()
            pltpu.make_async_copy(w16_h.at[pl.ds(0, CHUNK)], wv[b],
                                  sem_l[b]).wait()

        def multiply(b):
            rv, wvb = rows[b], wv[b]

            @pl.loop(0, CHUNK)
            def _(i):
                ws = wvb[i, :]
                for j in range(dh // 16):
                    sl = pl.ds(j * 16, 16)
                    rv[i, sl] = rv[i, sl] * ws

        def process(b):
            pltpu.sync_copy(t_h.at[isrc[b]], rows[b])
            multiply(b)
            pltpu.sync_copy(rows[b], acc.at[idst[b]], add=True)

        pltpu.sync_copy(z_h.at[pl.ds(s * RPS, RPS)],
                        acc.at[pl.ds(s * RPS, RPS)])
        plsc.subcore_barrier()

        start_loads(0, 0)

        @pl.loop(0, NCH, step=2)
        def _(kk):
            wait_loads(0)
            start_loads(1, kk + 1)
            process(0)
            wait_loads(1)

            @pl.when(kk + 2 < NCH)
            def _():
                start_loads(0, kk + 2)

            process(1)

        plsc.subcore_barrier()
        pltpu.sync_copy(acc.at[pl.ds(s * RPS, RPS)],
                        out_h.at[c, pl.ds(s * RPS, RPS)])

    return k


_CONV_KERNEL = _make_conv_kernel()


def _conv_call(table, src, dst, w16, zeros):
    return _CONV_KERNEL(table, src, dst, w16, zeros)


def _norms(do_ref, di_ref):
    deg_o = do_ref[0, :, 0:1] + do_ref[1, :, 0:1]
    deg_i = di_ref[0, :, 0:1] + di_ref[1, :, 0:1]
    ns = jnp.where(deg_o > 0, lax.rsqrt(jnp.maximum(deg_o, 1e-12)), 0.0)
    nd = jnp.where(deg_i > 0, lax.rsqrt(jnp.maximum(deg_i, 1e-12)), 0.0)
    return ns, nd


def _dot(a, b):
    return jnp.dot(a, b, preferred_element_type=jnp.float32,
                   precision=lax.Precision.HIGHEST)


def _t1_call(x, W1, Wr, br2, dego, degi):
    def body(x_ref, w1_ref, wr_ref, br_ref, do_ref, di_ref,
             g1a_ref, g1b_ref, res_ref, ns_ref, nd_ref):
        ns, nd = _norms(do_ref, di_ref)
        xb = x_ref[...]
        g = ns * _dot(xb, w1_ref[...])
        g1a_ref[...] = g[:, :128]
        g1b_ref[...] = g[:, 128:]
        res_ref[...] = _dot(xb, wr_ref[...]) + br_ref[...]
        ns_ref[...] = ns
        nd_ref[...] = nd

    grid = NN // _BR
    return pl.pallas_call(
        body,
        grid=(grid,),
        in_specs=[
            pl.BlockSpec((_BR, 256), lambda i: (i, 0)),
            pl.BlockSpec((256, 256), lambda i: (0, 0)),
            pl.BlockSpec((256, 64), lambda i: (0, 0)),
            pl.BlockSpec((1, 64), lambda i: (0, 0)),
            pl.BlockSpec((NC, _BR, 128), lambda i: (0, i, 0)),
            pl.BlockSpec((NC, _BR, 128), lambda i: (0, i, 0)),
        ],
        out_specs=[
            pl.BlockSpec((_BR, 128), lambda i: (i, 0)),
            pl.BlockSpec((_BR, 128), lambda i: (i, 0)),
            pl.BlockSpec((_BR, 64), lambda i: (i, 0)),
            pl.BlockSpec((_BR, 1), lambda i: (i, 0)),
            pl.BlockSpec((_BR, 1), lambda i: (i, 0)),
        ],
        out_shape=[
            jax.ShapeDtypeStruct((NN, 128), jnp.float32),
            jax.ShapeDtypeStruct((NN, 128), jnp.float32),
            jax.ShapeDtypeStruct((NN, 64), jnp.float32),
            jax.ShapeDtypeStruct((NN, 1), jnp.float32),
            jax.ShapeDtypeStruct((NN, 1), jnp.float32),
        ],
    )(x, W1, Wr, br2, dego, degi)


def _tmid_call(parts, b2, W, ns, nd, relu=True):
    nparts = len(parts)
    dprev = sum(p.shape[2] for p in parts)
    dout = W.shape[1]
    nouts = max(1, dout // 128)

    def body(*refs):
        p_refs = refs[:nparts]
        b_ref, w_ref, ns_ref, nd_ref = refs[nparts:nparts + 4]
        o_refs = refs[nparts + 4:]
        ns_v = ns_ref[...]
        nd_v = nd_ref[...]
        g = None
        col = 0
        for kk, p_ref in enumerate(p_refs):
            dh = p_ref.shape[2]
            h = nd_v * (p_ref[0] + p_ref[1]) + b_ref[:, col:col + dh]
            if relu:
                h = jnp.maximum(h, 0.0)
            contrib = _dot(h, w_ref[col:col + dh, :])
            g = contrib if g is None else g + contrib
            col += dh
        g = ns_v * g
        if dout < 128:
            g = jnp.concatenate([g, jnp.zeros((g.shape[0], 128 - dout),
                                              jnp.float32)], axis=1)
        if nouts == 1:
            o_refs[0][...] = g
        else:
            for kk in range(nouts):
                o_refs[kk][...] = g[:, kk * 128:(kk + 1) * 128]

    grid = NN // _BR
    in_specs = [pl.BlockSpec((NC, _BR, p.shape[2]), lambda i: (0, i, 0))
                for p in parts]
    in_specs += [
        pl.BlockSpec((1, dprev), lambda i: (0, 0)),
        pl.BlockSpec((dprev, dout), lambda i: (0, 0)),
        pl.BlockSpec((_BR, 1), lambda i: (i, 0)),
        pl.BlockSpec((_BR, 1), lambda i: (i, 0)),
    ]
    ow = 128
    out_specs = [pl.BlockSpec((_BR, ow), lambda i: (i, 0))] * nouts
    out_shape = [jax.ShapeDtypeStruct((NN, ow), jnp.float32)] * nouts
    res = pl.pallas_call(
        body, grid=(grid,), in_specs=in_specs, out_specs=out_specs,
        out_shape=out_shape,
    )(*parts, b2, W, ns, nd)
    return list(res)


def _t6_call(p5, b52, res, nd):
    def body(p_ref, b_ref, r_ref, nd_ref, o_ref):
        o_ref[...] = (nd_ref[...] * (p_ref[0][:, :64] + p_ref[1][:, :64])
                      + b_ref[...] + r_ref[...])

    grid = NN // _BR
    return pl.pallas_call(
        body,
        grid=(grid,),
        in_specs=[
            pl.BlockSpec((NC, _BR, 128), lambda i: (0, i, 0)),
            pl.BlockSpec((1, 64), lambda i: (0, 0)),
            pl.BlockSpec((_BR, 64), lambda i: (i, 0)),
            pl.BlockSpec((_BR, 1), lambda i: (i, 0)),
        ],
        out_specs=pl.BlockSpec((_BR, 64), lambda i: (i, 0)),
        out_shape=jax.ShapeDtypeStruct((NN, 64), jnp.float32),
    )(p5, b52, res, nd)


def kernel(x, edge_index, edge_weight, W1, b1, W2, b2, W3, b3, W4, b4,
           W5, b5, Wr, br):
    pad = NEP - NE
    src = jnp.concatenate([edge_index[0], jnp.zeros((pad,), jnp.int32)])
    dst = jnp.concatenate([edge_index[1], jnp.zeros((pad,), jnp.int32)])
    w16 = jnp.broadcast_to(
        jnp.concatenate([edge_weight, jnp.zeros((pad,), jnp.float32)])[:, None],
        (NEP, 16))

    ones = jnp.ones((NN, 128), jnp.float32)
    zeros = jnp.zeros((NNP, 128), jnp.float32)
    dego = _conv_call(ones, src, src, w16, zeros)
    degi = _conv_call(ones, dst, dst, w16, zeros)
    g1a, g1b, res, ns, nd = _t1_call(
        x, W1, Wr, br.reshape(1, 64), dego, degi)

    p1a = _conv_call(g1a, src, dst, w16, zeros)
    p1b = _conv_call(g1b, src, dst, w16, zeros)
    g2a, g2b = _tmid_call([p1a, p1b], b1.reshape(1, 256), W2, ns, nd)

    p2a = _conv_call(g2a, src, dst, w16, zeros)
    p2b = _conv_call(g2b, src, dst, w16, zeros)
    (g3,) = _tmid_call([p2a, p2b], b2.reshape(1, 256), W3, ns, nd)

    p3 = _conv_call(g3, src, dst, w16, zeros)
    (g4,) = _tmid_call([p3], b3.reshape(1, 128), W4, ns, nd)

    p4 = _conv_call(g4, src, dst, w16, zeros)
    (g5,) = _tmid_call([p4], b4.reshape(1, 128), W5, ns, nd)

    p5 = _conv_call(g5, src, dst, w16, zeros)
    return _t6_call(p5, b5.reshape(1, 64), res, nd)


# speedup vs baseline: 1.0509x; 1.0509x over previous
import jax
import jax.numpy as jnp
from jax import lax
from jax.experimental import pallas as pl
from jax.experimental.pallas import tpu as pltpu
from jax.experimental.pallas import tpu_sc as plsc

NN = 10000
NE = 160000
NC = 2
NS = 16
NWORK = NC * NS
NEP = 163840
EPW = NEP // NWORK
CHUNK = 64
NCH = EPW // CHUNK
NNP = 10240
RPS = NNP // NS

_BR = 2000


def _sc_mesh():
    return plsc.VectorSubcoreMesh(core_axis_name="c", subcore_axis_name="s")


def _make_conv_kernel():
    dh = 128
    out_t = jax.ShapeDtypeStruct((NC, NNP, dh), jnp.float32)

    @pl.kernel(out_type=out_t, mesh=_sc_mesh(),
               scratch_types=[pltpu.VMEM((CHUNK,), jnp.int32),
                              pltpu.VMEM((CHUNK,), jnp.int32),
                              pltpu.VMEM((CHUNK,), jnp.int32),
                              pltpu.VMEM((CHUNK,), jnp.int32),
                              pltpu.VMEM((CHUNK,), jnp.int32),
                              pltpu.VMEM((CHUNK,), jnp.int32),
                              pltpu.VMEM((CHUNK, 16), jnp.float32),
                              pltpu.VMEM((CHUNK, 16), jnp.float32),
                              pltpu.VMEM((CHUNK, dh), jnp.float32),
                              pltpu.VMEM((CHUNK, dh), jnp.float32),
                              pltpu.VMEM_SHARED((NNP, dh), jnp.float32),
                              pltpu.SemaphoreType.DMA,
                              pltpu.SemaphoreType.DMA,
                              pltpu.SemaphoreType.DMA,
                              pltpu.SemaphoreType.DMA,
                              pltpu.SemaphoreType.DMA,
                              pltpu.SemaphoreType.DMA,
                              pltpu.SemaphoreType.DMA,
                              pltpu.SemaphoreType.DMA])
    def k(t_h, src_h, dst_h, w16_h, z_h, out_h,
          isrc0, isrc1, idst0, idst1, idst2, idst3,
          wv0, wv1, rows0, rows1, acc,
          sl0, sl1, sl2, sl3, sg0, sg1, ss0, ss1):
        c = lax.axis_index("c")
        s = lax.axis_index("s")
        wid = c * NS + s
        isrc = (isrc0, isrc1)
        idst = (idst0, idst1, idst2, idst3)
        wv = (wv0, wv1)
        rows = (rows0, rows1)
        sem_l = (sl0, sl1, sl2, sl3)
        sem_g = (sg0, sg1)
        sem_s = (ss0, ss1)

        def start_loads(u, kk):
            b, i = u % 2, u % 4
            base = wid * EPW + kk * CHUNK
            pltpu.async_copy(src_h.at[pl.ds(base, CHUNK)], isrc[b], sem_l[i])
            pltpu.async_copy(dst_h.at[pl.ds(base, CHUNK)], idst[i], sem_l[i])
            pltpu.async_copy(w16_h.at[pl.ds(base, CHUNK)], wv[b], sem_l[i])

        def wait_loads(u):
            b, i = u % 2, u % 4
            pltpu.make_async_copy(src_h.at[pl.ds(0, CHUNK)], isrc[b],
                                  sem_l[i]).wait()
            pltpu.make_async_copy(dst_h.at[pl.ds(0, CHUNK)], idst[i],
                                  sem_l[i]).wait()
            pltpu.make_async_copy(w16_h.at[pl.ds(0, CHUNK)], wv[b],
                                  sem_l[i]).wait()

        def multiply(b, i):
            rv, wvb = rows[b], wv[b]

            @pl.loop(0, CHUNK)
            def _(ii):
                ws = wvb[ii, :]
                for j in range(dh // 16):
                    sl = pl.ds(j * 16, 16)
                    rv[ii, sl] = rv[ii, sl] * ws

        def start_gather(b, i):
            pltpu.async_copy(t_h.at[isrc[b]], rows[b], sem_g[b])

        def wait_gather(b, i):
            pltpu.make_async_copy(t_h.at[isrc[b]], rows[b], sem_g[b]).wait()

        def start_scatter(b, i):
            pltpu.async_copy(rows[b], acc.at[idst[i]], sem_s[b], add=True)

        def wait_scatter(b, i):
            pltpu.make_async_copy(rows[b], acc.at[idst[i]],
                                  sem_s[b]).wait()

        pltpu.sync_copy(z_h.at[pl.ds(s * RPS, RPS)],
                        acc.at[pl.ds(s * RPS, RPS)])
        plsc.subcore_barrier()

        start_loads(0, 0)
        start_loads(1, 1)

        @pl.loop(0, NCH, step=4)
        def _(kk):
            for u in range(4):
                b = u % 2
                i = u % 4
                cc = kk + u

                @pl.when(cc >= 2)
                def _():
                    wait_scatter(b, (u + 2) % 4)

                wait_loads(u)
                start_gather(b, i)
                wait_gather(b, i)
                multiply(b, i)
                start_scatter(b, i)

                @pl.when(cc + 2 < NCH)
                def _():
                    start_loads(u + 2, cc + 2)

        wait_scatter(0, 2)
        wait_scatter(1, 3)
        plsc.subcore_barrier()
        pltpu.sync_copy(acc.at[pl.ds(s * RPS, RPS)],
                        out_h.at[c, pl.ds(s * RPS, RPS)])

    return k


_CONV_KERNEL = _make_conv_kernel()


def _conv_call(table, src, dst, w16, zeros):
    return _CONV_KERNEL(table, src, dst, w16, zeros)


def _norms(do_ref, di_ref):
    deg_o = do_ref[0, :, 0:1] + do_ref[1, :, 0:1]
    deg_i = di_ref[0, :, 0:1] + di_ref[1, :, 0:1]
    ns = jnp.where(deg_o > 0, lax.rsqrt(jnp.maximum(deg_o, 1e-12)), 0.0)
    nd = jnp.where(deg_i > 0, lax.rsqrt(jnp.maximum(deg_i, 1e-12)), 0.0)
    return ns, nd


def _dot(a, b):
    return jnp.dot(a, b, preferred_element_type=jnp.float32,
                   precision=lax.Precision.HIGHEST)


def _t1_call(x, W1, Wr, br2, dego, degi):
    def body(x_ref, w1_ref, wr_ref, br_ref, do_ref, di_ref,
             g1a_ref, g1b_ref, res_ref, ns_ref, nd_ref):
        ns, nd = _norms(do_ref, di_ref)
        xb = x_ref[...]
        g = ns * _dot(xb, w1_ref[...])
        g1a_ref[...] = g[:, :128]
        g1b_ref[...] = g[:, 128:]
        res_ref[...] = _dot(xb, wr_ref[...]) + br_ref[...]
        ns_ref[...] = ns
        nd_ref[...] = nd

    grid = NN // _BR
    return pl.pallas_call(
        body,
        grid=(grid,),
        in_specs=[
            pl.BlockSpec((_BR, 256), lambda i: (i, 0)),
            pl.BlockSpec((256, 256), lambda i: (0, 0)),
            pl.BlockSpec((256, 64), lambda i: (0, 0)),
            pl.BlockSpec((1, 64), lambda i: (0, 0)),
            pl.BlockSpec((NC, _BR, 128), lambda i: (0, i, 0)),
            pl.BlockSpec((NC, _BR, 128), lambda i: (0, i, 0)),
        ],
        out_specs=[
            pl.BlockSpec((_BR, 128), lambda i: (i, 0)),
            pl.BlockSpec((_BR, 128), lambda i: (i, 0)),
            pl.BlockSpec((_BR, 64), lambda i: (i, 0)),
            pl.BlockSpec((_BR, 1), lambda i: (i, 0)),
            pl.BlockSpec((_BR, 1), lambda i: (i, 0)),
        ],
        out_shape=[
            jax.ShapeDtypeStruct((NN, 128), jnp.float32),
            jax.ShapeDtypeStruct((NN, 128), jnp.float32),
            jax.ShapeDtypeStruct((NN, 64), jnp.float32),
            jax.ShapeDtypeStruct((NN, 1), jnp.float32),
            jax.ShapeDtypeStruct((NN, 1), jnp.float32),
        ],
    )(x, W1, Wr, br2, dego, degi)


def _tmid_call(parts, b2, W, ns, nd, relu=True):
    nparts = len(parts)
    dprev = sum(p.shape[2] for p in parts)
    dout = W.shape[1]
    nouts = max(1, dout // 128)

    def body(*refs):
        p_refs = refs[:nparts]
        b_ref, w_ref, ns_ref, nd_ref = refs[nparts:nparts + 4]
        o_refs = refs[nparts + 4:]
        ns_v = ns_ref[...]
        nd_v = nd_ref[...]
        g = None
        col = 0
        for kk, p_ref in enumerate(p_refs):
            dh = p_ref.shape[2]
            h = nd_v * (p_ref[0] + p_ref[1]) + b_ref[:, col:col + dh]
            if relu:
                h = jnp.maximum(h, 0.0)
            contrib = _dot(h, w_ref[col:col + dh, :])
            g = contrib if g is None else g + contrib
            col += dh
        g = ns_v * g
        if dout < 128:
            g = jnp.concatenate([g, jnp.zeros((g.shape[0], 128 - dout),
                                              jnp.float32)], axis=1)
        if nouts == 1:
            o_refs[0][...] = g
        else:
            for kk in range(nouts):
                o_refs[kk][...] = g[:, kk * 128:(kk + 1) * 128]

    grid = NN // _BR
    in_specs = [pl.BlockSpec((NC, _BR, p.shape[2]), lambda i: (0, i, 0))
                for p in parts]
    in_specs += [
        pl.BlockSpec((1, dprev), lambda i: (0, 0)),
        pl.BlockSpec((dprev, dout), lambda i: (0, 0)),
        pl.BlockSpec((_BR, 1), lambda i: (i, 0)),
        pl.BlockSpec((_BR, 1), lambda i: (i, 0)),
    ]
    ow = 128
    out_specs = [pl.BlockSpec((_BR, ow), lambda i: (i, 0))] * nouts
    out_shape = [jax.ShapeDtypeStruct((NN, ow), jnp.float32)] * nouts
    res = pl.pallas_call(
        body, grid=(grid,), in_specs=in_specs, out_specs=out_specs,
        out_shape=out_shape,
    )(*parts, b2, W, ns, nd)
    return list(res)


def _t6_call(p5, b52, res, nd):
    def body(p_ref, b_ref, r_ref, nd_ref, o_ref):
        o_ref[...] = (nd_ref[...] * (p_ref[0][:, :64] + p_ref[1][:, :64])
                      + b_ref[...] + r_ref[...])

    grid = NN // _BR
    return pl.pallas_call(
        body,
        grid=(grid,),
        in_specs=[
            pl.BlockSpec((NC, _BR, 128), lambda i: (0, i, 0)),
            pl.BlockSpec((1, 64), lambda i: (0, 0)),
            pl.BlockSpec((_BR, 64), lambda i: (i, 0)),
            pl.BlockSpec((_BR, 1), lambda i: (i, 0)),
        ],
        out_specs=pl.BlockSpec((_BR, 64), lambda i: (i, 0)),
        out_shape=jax.ShapeDtypeStruct((NN, 64), jnp.float32),
    )(p5, b52, res, nd)


def kernel(x, edge_index, edge_weight, W1, b1, W2, b2, W3, b3, W4, b4,
           W5, b5, Wr, br):
    pad = NEP - NE
    src = jnp.concatenate([edge_index[0], jnp.zeros((pad,), jnp.int32)])
    dst = jnp.concatenate([edge_index[1], jnp.zeros((pad,), jnp.int32)])
    w16 = jnp.broadcast_to(
        jnp.concatenate([edge_weight, jnp.zeros((pad,), jnp.float32)])[:, None],
        (NEP, 16))

    ones = jnp.ones((NN, 128), jnp.float32)
    zeros = jnp.zeros((NNP, 128), jnp.float32)
    dego = _conv_call(ones, src, src, w16, zeros)
    degi = _conv_call(ones, dst, dst, w16, zeros)
    g1a, g1b, res, ns, nd = _t1_call(
        x, W1, Wr, br.reshape(1, 64), dego, degi)

    p1a = _conv_call(g1a, src, dst, w16, zeros)
    p1b = _conv_call(g1b, src, dst, w16, zeros)
    g2a, g2b = _tmid_call([p1a, p1b], b1.reshape(1, 256), W2, ns, nd)

    p2a = _conv_call(g2a, src, dst, w16, zeros)
    p2b = _conv_call(g2b, src, dst, w16, zeros)
    (g3,) = _tmid_call([p2a, p2b], b2.reshape(1, 256), W3, ns, nd)

    p3 = _conv_call(g3, src, dst, w16, zeros)
    (g4,) = _tmid_call([p3], b3.reshape(1, 128), W4, ns, nd)

    p4 = _conv_call(g4, src, dst, w16, zeros)
    (g5,) = _tmid_call([p4], b4.reshape(1, 128), W5, ns, nd)

    p5 = _conv_call(g5, src, dst, w16, zeros)
    return _t6_call(p5, b5.reshape(1, 64), res, nd)


# speedup vs baseline: 1.1319x; 1.0772x over previous
import jax
import jax.numpy as jnp
from jax import lax
from jax.experimental import pallas as pl
from jax.experimental.pallas import tpu as pltpu
from jax.experimental.pallas import tpu_sc as plsc

NN = 10000
NE = 160000
NC = 2
NS = 16
NWORK = NC * NS
NEP = 163840
EPW = NEP // NWORK
CHUNK = 64
NCH = EPW // CHUNK
NNP = 10240
RPS = NNP // NS

_BR = 2000


def _sc_mesh():
    return plsc.VectorSubcoreMesh(core_axis_name="c", subcore_axis_name="s")


def _make_conv_kernel():
    dh = 128
    out_t = jax.ShapeDtypeStruct((NC, NNP, dh), jnp.float32)

    @pl.kernel(out_type=out_t, mesh=_sc_mesh(),
               scratch_types=[pltpu.VMEM((CHUNK,), jnp.int32),
                              pltpu.VMEM((CHUNK,), jnp.int32),
                              pltpu.VMEM((CHUNK,), jnp.int32),
                              pltpu.VMEM((CHUNK,), jnp.int32),
                              pltpu.VMEM((CHUNK,), jnp.int32),
                              pltpu.VMEM((CHUNK,), jnp.int32),
                              pltpu.VMEM((CHUNK, 16), jnp.float32),
                              pltpu.VMEM((CHUNK, 16), jnp.float32),
                              pltpu.VMEM((CHUNK, dh), jnp.float32),
                              pltpu.VMEM((CHUNK, dh), jnp.float32),
                              pltpu.VMEM_SHARED((NNP, dh), jnp.float32),
                              pltpu.SemaphoreType.DMA,
                              pltpu.SemaphoreType.DMA,
                              pltpu.SemaphoreType.DMA,
                              pltpu.SemaphoreType.DMA,
                              pltpu.SemaphoreType.DMA,
                              pltpu.SemaphoreType.DMA,
                              pltpu.SemaphoreType.DMA,
                              pltpu.SemaphoreType.DMA])
    def k(t_h, src_h, dst_h, w16_h, z_h, out_h,
          isrc0, isrc1, idst0, idst1, idst2, idst3,
          wv0, wv1, rows0, rows1, acc,
          sl0, sl1, sl2, sl3, sg0, sg1, ss0, ss1):
        c = lax.axis_index("c")
        s = lax.axis_index("s")
        wid = c * NS + s
        isrc = (isrc0, isrc1)
        idst = (idst0, idst1, idst2, idst3)
        wv = (wv0, wv1)
        rows = (rows0, rows1)
        sem_l = (sl0, sl1, sl2, sl3)
        sem_g = (sg0, sg1)
        sem_s = (ss0, ss1)

        def start_loads(u, kk):
            b, i = u % 2, u % 4
            base = wid * EPW + kk * CHUNK
            pltpu.async_copy(src_h.at[pl.ds(base, CHUNK)], isrc[b], sem_l[i])
            pltpu.async_copy(dst_h.at[pl.ds(base, CHUNK)], idst[i], sem_l[i])
            pltpu.async_copy(w16_h.at[pl.ds(base, CHUNK)], wv[b], sem_l[i])

        def wait_loads(u):
            b, i = u % 2, u % 4
            pltpu.make_async_copy(src_h.at[pl.ds(0, CHUNK)], isrc[b],
                                  sem_l[i]).wait()
            pltpu.make_async_copy(dst_h.at[pl.ds(0, CHUNK)], idst[i],
                                  sem_l[i]).wait()
            pltpu.make_async_copy(w16_h.at[pl.ds(0, CHUNK)], wv[b],
                                  sem_l[i]).wait()

        def multiply(b, i):
            rv, wvb = rows[b], wv[b]

            @pl.loop(0, CHUNK)
            def _(ii):
                ws = wvb[ii, :]
                for j in range(dh // 16):
                    sl = pl.ds(j * 16, 16)
                    rv[ii, sl] = rv[ii, sl] * ws

        def start_gather(b, i):
            pltpu.async_copy(t_h.at[isrc[b]], rows[b], sem_g[b])

        def wait_gather(b, i):
            pltpu.make_async_copy(t_h.at[isrc[b]], rows[b], sem_g[b]).wait()

        def start_scatter(b, i):
            pltpu.async_copy(rows[b], acc.at[idst[i]], sem_s[b], add=True)

        def wait_scatter(b, i):
            pltpu.make_async_copy(rows[b], acc.at[idst[i]],
                                  sem_s[b]).wait()

        pltpu.sync_copy(z_h.at[pl.ds(s * RPS, RPS)],
                        acc.at[pl.ds(s * RPS, RPS)])
        plsc.subcore_barrier()

        start_loads(0, 0)
        start_loads(1, 1)
        wait_loads(0)
        start_gather(0, 0)

        @pl.loop(0, NCH, step=4)
        def _(kk):
            for u in range(4):
                b = u % 2
                i = u % 4
                cc = kk + u
                wait_gather(b, i)

                @pl.when(cc + 1 < NCH)
                def _():
                    wait_loads(u + 1)

                @pl.when(cc >= 1)
                def _():
                    wait_scatter(1 - b, (u + 3) % 4)

                @pl.when(cc + 1 < NCH)
                def _():
                    start_gather(1 - b, (u + 1) % 4)

                multiply(b, i)
                start_scatter(b, i)

                @pl.when(cc + 2 < NCH)
                def _():
                    start_loads(u + 2, cc + 2)

        wait_scatter(1, 3)
        plsc.subcore_barrier()
        pltpu.sync_copy(acc.at[pl.ds(s * RPS, RPS)],
                        out_h.at[c, pl.ds(s * RPS, RPS)])

    return k


_CONV_KERNEL = _make_conv_kernel()


def _conv_call(table, src, dst, w16, zeros):
    return _CONV_KERNEL(table, src, dst, w16, zeros)


def _norms(do_ref, di_ref):
    deg_o = do_ref[0, :, 0:1] + do_ref[1, :, 0:1]
    deg_i = di_ref[0, :, 0:1] + di_ref[1, :, 0:1]
    ns = jnp.where(deg_o > 0, lax.rsqrt(jnp.maximum(deg_o, 1e-12)), 0.0)
    nd = jnp.where(deg_i > 0, lax.rsqrt(jnp.maximum(deg_i, 1e-12)), 0.0)
    return ns, nd


def _dot(a, b):
    return jnp.dot(a, b, preferred_element_type=jnp.float32,
                   precision=lax.Precision.HIGHEST)


def _t1_call(x, W1, Wr, br2, dego, degi):
    def body(x_ref, w1_ref, wr_ref, br_ref, do_ref, di_ref,
             g1a_ref, g1b_ref, res_ref, ns_ref, nd_ref):
        ns, nd = _norms(do_ref, di_ref)
        xb = x_ref[...]
        g = ns * _dot(xb, w1_ref[...])
        g1a_ref[...] = g[:, :128]
        g1b_ref[...] = g[:, 128:]
        res_ref[...] = _dot(xb, wr_ref[...]) + br_ref[...]
        ns_ref[...] = ns
        nd_ref[...] = nd

    grid = NN // _BR
    return pl.pallas_call(
        body,
        grid=(grid,),
        in_specs=[
            pl.BlockSpec((_BR, 256), lambda i: (i, 0)),
            pl.BlockSpec((256, 256), lambda i: (0, 0)),
            pl.BlockSpec((256, 64), lambda i: (0, 0)),
            pl.BlockSpec((1, 64), lambda i: (0, 0)),
            pl.BlockSpec((NC, _BR, 128), lambda i: (0, i, 0)),
            pl.BlockSpec((NC, _BR, 128), lambda i: (0, i, 0)),
        ],
        out_specs=[
            pl.BlockSpec((_BR, 128), lambda i: (i, 0)),
            pl.BlockSpec((_BR, 128), lambda i: (i, 0)),
            pl.BlockSpec((_BR, 64), lambda i: (i, 0)),
            pl.BlockSpec((_BR, 1), lambda i: (i, 0)),
            pl.BlockSpec((_BR, 1), lambda i: (i, 0)),
        ],
        out_shape=[
            jax.ShapeDtypeStruct((NN, 128), jnp.float32),
            jax.ShapeDtypeStruct((NN, 128), jnp.float32),
            jax.ShapeDtypeStruct((NN, 64), jnp.float32),
            jax.ShapeDtypeStruct((NN, 1), jnp.float32),
            jax.ShapeDtypeStruct((NN, 1), jnp.float32),
        ],
    )(x, W1, Wr, br2, dego, degi)


def _tmid_call(parts, b2, W, ns, nd, relu=True):
    nparts = len(parts)
    dprev = sum(p.shape[2] for p in parts)
    dout = W.shape[1]
    nouts = max(1, dout // 128)

    def body(*refs):
        p_refs = refs[:nparts]
        b_ref, w_ref, ns_ref, nd_ref = refs[nparts:nparts + 4]
        o_refs = refs[nparts + 4:]
        ns_v = ns_ref[...]
        nd_v = nd_ref[...]
        g = None
        col = 0
        for kk, p_ref in enumerate(p_refs):
            dh = p_ref.shape[2]
            h = nd_v * (p_ref[0] + p_ref[1]) + b_ref[:, col:col + dh]
            if relu:
                h = jnp.maximum(h, 0.0)
            contrib = _dot(h, w_ref[col:col + dh, :])
            g = contrib if g is None else g + contrib
            col += dh
        g = ns_v * g
        if dout < 128:
            g = jnp.concatenate([g, jnp.zeros((g.shape[0], 128 - dout),
                                              jnp.float32)], axis=1)
        if nouts == 1:
            o_refs[0][...] = g
        else:
            for kk in range(nouts):
                o_refs[kk][...] = g[:, kk * 128:(kk + 1) * 128]

    grid = NN // _BR
    in_specs = [pl.BlockSpec((NC, _BR, p.shape[2]), lambda i: (0, i, 0))
                for p in parts]
    in_specs += [
        pl.BlockSpec((1, dprev), lambda i: (0, 0)),
        pl.BlockSpec((dprev, dout), lambda i: (0, 0)),
        pl.BlockSpec((_BR, 1), lambda i: (i, 0)),
        pl.BlockSpec((_BR, 1), lambda i: (i, 0)),
    ]
    ow = 128
    out_specs = [pl.BlockSpec((_BR, ow), lambda i: (i, 0))] * nouts
    out_shape = [jax.ShapeDtypeStruct((NN, ow), jnp.float32)] * nouts
    res = pl.pallas_call(
        body, grid=(grid,), in_specs=in_specs, out_specs=out_specs,
        out_shape=out_shape,
    )(*parts, b2, W, ns, nd)
    return list(res)


def _t6_call(p5, b52, res, nd):
    def body(p_ref, b_ref, r_ref, nd_ref, o_ref):
        o_ref[...] = (nd_ref[...] * (p_ref[0][:, :64] + p_ref[1][:, :64])
                      + b_ref[...] + r_ref[...])

    grid = NN // _BR
    return pl.pallas_call(
        body,
        grid=(grid,),
        in_specs=[
            pl.BlockSpec((NC, _BR, 128), lambda i: (0, i, 0)),
            pl.BlockSpec((1, 64), lambda i: (0, 0)),
            pl.BlockSpec((_BR, 64), lambda i: (i, 0)),
            pl.BlockSpec((_BR, 1), lambda i: (i, 0)),
        ],
        out_specs=pl.BlockSpec((_BR, 64), lambda i: (i, 0)),
        out_shape=jax.ShapeDtypeStruct((NN, 64), jnp.float32),
    )(p5, b52, res, nd)


def kernel(x, edge_index, edge_weight, W1, b1, W2, b2, W3, b3, W4, b4,
           W5, b5, Wr, br):
    pad = NEP - NE
    src = jnp.concatenate([edge_index[0], jnp.zeros((pad,), jnp.int32)])
    dst = jnp.concatenate([edge_index[1], jnp.zeros((pad,), jnp.int32)])
    w16 = jnp.broadcast_to(
        jnp.concatenate([edge_weight, jnp.zeros((pad,), jnp.float32)])[:, None],
        (NEP, 16))

    ones = jnp.ones((NN, 128), jnp.float32)
    zeros = jnp.zeros((NNP, 128), jnp.float32)
    dego = _conv_call(ones, src, src, w16, zeros)
    degi = _conv_call(ones, dst, dst, w16, zeros)
    g1a, g1b, res, ns, nd = _t1_call(
        x, W1, Wr, br.reshape(1, 64), dego, degi)

    p1a = _conv_call(g1a, src, dst, w16, zeros)
    p1b = _conv_call(g1b, src, dst, w16, zeros)
    g2a, g2b = _tmid_call([p1a, p1b], b1.reshape(1, 256), W2, ns, nd)

    p2a = _conv_call(g2a, src, dst, w16, zeros)
    p2b = _conv_call(g2b, src, dst, w16, zeros)
    (g3,) = _tmid_call([p2a, p2b], b2.reshape(1, 256), W3, ns, nd)

    p3 = _conv_call(g3, src, dst, w16, zeros)
    (g4,) = _tmid_call([p3], b3.reshape(1, 128), W4, ns, nd)

    p4 = _conv_call(g4, src, dst, w16, zeros)
    (g5,) = _tmid_call([p4], b4.reshape(1, 128), W5, ns, nd)

    p5 = _conv_call(g5, src, dst, w16, zeros)
    return _t6_call(p5, b5.reshape(1, 64), res, nd)


# speedup vs baseline: 1.1323x; 1.0003x over previous
import jax
import jax.numpy as jnp
from jax import lax
from jax.experimental import pallas as pl
from jax.experimental.pallas import tpu as pltpu
from jax.experimental.pallas import tpu_sc as plsc

NN = 10000
NE = 160000
NC = 2
NS = 16
NWORK = NC * NS
NEP = 163840
EPW = NEP // NWORK
CHUNK = 64
NCH = EPW // CHUNK
NNP = 10240
RPS = NNP // NS

_BR = 2000


def _sc_mesh():
    return plsc.VectorSubcoreMesh(core_axis_name="c", subcore_axis_name="s")


def _make_conv_kernel():
    dh = 128
    out_t = jax.ShapeDtypeStruct((NC, NNP, dh), jnp.float32)

    @pl.kernel(out_type=out_t, mesh=_sc_mesh(),
               scratch_types=[pltpu.VMEM((CHUNK,), jnp.int32),
                              pltpu.VMEM((CHUNK,), jnp.int32),
                              pltpu.VMEM((CHUNK,), jnp.int32),
                              pltpu.VMEM((CHUNK,), jnp.int32),
                              pltpu.VMEM((CHUNK,), jnp.int32),
                              pltpu.VMEM((CHUNK,), jnp.int32),
                              pltpu.VMEM((CHUNK, 16), jnp.float32),
                              pltpu.VMEM((CHUNK, 16), jnp.float32),
                              pltpu.VMEM((CHUNK, dh), jnp.float32),
                              pltpu.VMEM((CHUNK, dh), jnp.float32),
                              pltpu.VMEM_SHARED((NNP, dh), jnp.float32),
                              pltpu.SemaphoreType.DMA,
                              pltpu.SemaphoreType.DMA,
                              pltpu.SemaphoreType.DMA,
                              pltpu.SemaphoreType.DMA,
                              pltpu.SemaphoreType.DMA,
                              pltpu.SemaphoreType.DMA,
                              pltpu.SemaphoreType.DMA,
                              pltpu.SemaphoreType.DMA])
    def k(t_h, src_h, dst_h, w16_h, z_h, out_h,
          isrc0, isrc1, idst0, idst1, idst2, idst3,
          wv0, wv1, rows0, rows1, acc,
          sl0, sl1, sl2, sl3, sg0, sg1, ss0, ss1):
        c = lax.axis_index("c")
        s = lax.axis_index("s")
        wid = c * NS + s
        isrc = (isrc0, isrc1)
        idst = (idst0, idst1, idst2, idst3)
        wv = (wv0, wv1)
        rows = (rows0, rows1)
        sem_l = (sl0, sl1, sl2, sl3)
        sem_g = (sg0, sg1)
        sem_s = (ss0, ss1)

        def start_loads(u, kk):
            b, i = u % 2, u % 4
            base = wid * EPW + kk * CHUNK
            pltpu.async_copy(src_h.at[pl.ds(base, CHUNK)], isrc[b], sem_l[i])
            pltpu.async_copy(dst_h.at[pl.ds(base, CHUNK)], idst[i], sem_l[i])
            pltpu.async_copy(w16_h.at[pl.ds(base, CHUNK)], wv[b], sem_l[i])

        def wait_loads(u):
            b, i = u % 2, u % 4
            pltpu.make_async_copy(src_h.at[pl.ds(0, CHUNK)], isrc[b],
                                  sem_l[i]).wait()
            pltpu.make_async_copy(dst_h.at[pl.ds(0, CHUNK)], idst[i],
                                  sem_l[i]).wait()
            pltpu.make_async_copy(w16_h.at[pl.ds(0, CHUNK)], wv[b],
                                  sem_l[i]).wait()

        def multiply(b, i):
            rv, wvb = rows[b], wv[b]

            @pl.loop(0, CHUNK, step=4)
            def _(ii):
                ws = [wvb[ii + e, :] for e in range(4)]
                for j in range(dh // 16):
                    sl = pl.ds(j * 16, 16)
                    for e in range(4):
                        rv[ii + e, sl] = rv[ii + e, sl] * ws[e]

        def start_gather(b, i):
            pltpu.async_copy(t_h.at[isrc[b]], rows[b], sem_g[b])

        def wait_gather(b, i):
            pltpu.make_async_copy(t_h.at[isrc[b]], rows[b], sem_g[b]).wait()

        def start_scatter(b, i):
            pltpu.async_copy(rows[b], acc.at[idst[i]], sem_s[b], add=True)

        def wait_scatter(b, i):
            pltpu.make_async_copy(rows[b], acc.at[idst[i]],
                                  sem_s[b]).wait()

        pltpu.sync_copy(z_h.at[pl.ds(s * RPS, RPS)],
                        acc.at[pl.ds(s * RPS, RPS)])
        plsc.subcore_barrier()

        start_loads(0, 0)
        start_loads(1, 1)
        wait_loads(0)
        start_gather(0, 0)

        @pl.loop(0, NCH, step=4)
        def _(kk):
            for u in range(4):
                b = u % 2
                i = u % 4
                cc = kk + u
                wait_gather(b, i)

                @pl.when(cc + 1 < NCH)
                def _():
                    wait_loads(u + 1)

                @pl.when(cc >= 1)
                def _():
                    wait_scatter(1 - b, (u + 3) % 4)

                @pl.when(cc + 1 < NCH)
                def _():
                    start_gather(1 - b, (u + 1) % 4)

                multiply(b, i)
                start_scatter(b, i)

                @pl.when(cc + 2 < NCH)
                def _():
                    start_loads(u + 2, cc + 2)

        wait_scatter(1, 3)
        plsc.subcore_barrier()
        pltpu.sync_copy(acc.at[pl.ds(s * RPS, RPS)],
                        out_h.at[c, pl.ds(s * RPS, RPS)])

    return k


_CONV_KERNEL = _make_conv_kernel()


def _conv_call(table, src, dst, w16, zeros):
    return _CONV_KERNEL(table, src, dst, w16, zeros)


def _norms(do_ref, di_ref):
    deg_o = do_ref[0, :, 0:1] + do_ref[1, :, 0:1]
    deg_i = di_ref[0, :, 0:1] + di_ref[1, :, 0:1]
    ns = jnp.where(deg_o > 0, lax.rsqrt(jnp.maximum(deg_o, 1e-12)), 0.0)
    nd = jnp.where(deg_i > 0, lax.rsqrt(jnp.maximum(deg_i, 1e-12)), 0.0)
    return ns, nd


def _dot(a, b):
    return jnp.dot(a, b, preferred_element_type=jnp.float32,
                   precision=lax.Precision.HIGHEST)


def _t1_call(x, W1, Wr, br2, dego, degi):
    def body(x_ref, w1_ref, wr_ref, br_ref, do_ref, di_ref,
             g1a_ref, g1b_ref, res_ref, ns_ref, nd_ref):
        ns, nd = _norms(do_ref, di_ref)
        xb = x_ref[...]
        g = ns * _dot(xb, w1_ref[...])
        g1a_ref[...] = g[:, :128]
        g1b_ref[...] = g[:, 128:]
        res_ref[...] = _dot(xb, wr_ref[...]) + br_ref[...]
        ns_ref[...] = ns
        nd_ref[...] = nd

    grid = NN // _BR
    return pl.pallas_call(
        body,
        grid=(grid,),
        in_specs=[
            pl.BlockSpec((_BR, 256), lambda i: (i, 0)),
            pl.BlockSpec((256, 256), lambda i: (0, 0)),
            pl.BlockSpec((256, 64), lambda i: (0, 0)),
            pl.BlockSpec((1, 64), lambda i: (0, 0)),
            pl.BlockSpec((NC, _BR, 128), lambda i: (0, i, 0)),
            pl.BlockSpec((NC, _BR, 128), lambda i: (0, i, 0)),
        ],
        out_specs=[
            pl.BlockSpec((_BR, 128), lambda i: (i, 0)),
            pl.BlockSpec((_BR, 128), lambda i: (i, 0)),
            pl.BlockSpec((_BR, 64), lambda i: (i, 0)),
            pl.BlockSpec((_BR, 1), lambda i: (i, 0)),
            pl.BlockSpec((_BR, 1), lambda i: (i, 0)),
        ],
        out_shape=[
            jax.ShapeDtypeStruct((NN, 128), jnp.float32),
            jax.ShapeDtypeStruct((NN, 128), jnp.float32),
            jax.ShapeDtypeStruct((NN, 64), jnp.float32),
            jax.ShapeDtypeStruct((NN, 1), jnp.float32),
            jax.ShapeDtypeStruct((NN, 1), jnp.float32),
        ],
    )(x, W1, Wr, br2, dego, degi)


def _tmid_call(parts, b2, W, ns, nd, relu=True):
    nparts = len(parts)
    dprev = sum(p.shape[2] for p in parts)
    dout = W.shape[1]
    nouts = max(1, dout // 128)

    def body(*refs):
        p_refs = refs[:nparts]
        b_ref, w_ref, ns_ref, nd_ref = refs[nparts:nparts + 4]
        o_refs = refs[nparts + 4:]
        ns_v = ns_ref[...]
        nd_v = nd_ref[...]
        g = None
        col = 0
        for kk, p_ref in enumerate(p_refs):
            dh = p_ref.shape[2]
            h = nd_v * (p_ref[0] + p_ref[1]) + b_ref[:, col:col + dh]
            if relu:
                h = jnp.maximum(h, 0.0)
            contrib = _dot(h, w_ref[col:col + dh, :])
            g = contrib if g is None else g + contrib
            col += dh
        g = ns_v * g
        if dout < 128:
            g = jnp.concatenate([g, jnp.zeros((g.shape[0], 128 - dout),
                                              jnp.float32)], axis=1)
        if nouts == 1:
            o_refs[0][...] = g
        else:
            for kk in range(nouts):
                o_refs[kk][...] = g[:, kk * 128:(kk + 1) * 128]

    grid = NN // _BR
    in_specs = [pl.BlockSpec((NC, _BR, p.shape[2]), lambda i: (0, i, 0))
                for p in parts]
    in_specs += [
        pl.BlockSpec((1, dprev), lambda i: (0, 0)),
        pl.BlockSpec((dprev, dout), lambda i: (0, 0)),
        pl.BlockSpec((_BR, 1), lambda i: (i, 0)),
        pl.BlockSpec((_BR, 1), lambda i: (i, 0)),
    ]
    ow = 128
    out_specs = [pl.BlockSpec((_BR, ow), lambda i: (i, 0))] * nouts
    out_shape = [jax.ShapeDtypeStruct((NN, ow), jnp.float32)] * nouts
    res = pl.pallas_call(
        body, grid=(grid,), in_specs=in_specs, out_specs=out_specs,
        out_shape=out_shape,
    )(*parts, b2, W, ns, nd)
    return list(res)


def _t6_call(p5, b52, res, nd):
    def body(p_ref, b_ref, r_ref, nd_ref, o_ref):
        o_ref[...] = (nd_ref[...] * (p_ref[0][:, :64] + p_ref[1][:, :64])
                      + b_ref[...] + r_ref[...])

    grid = NN // _BR
    return pl.pallas_call(
        body,
        grid=(grid,),
        in_specs=[
            pl.BlockSpec((NC, _BR, 128), lambda i: (0, i, 0)),
            pl.BlockSpec((1, 64), lambda i: (0, 0)),
            pl.BlockSpec((_BR, 64), lambda i: (i, 0)),
            pl.BlockSpec((_BR, 1), lambda i: (i, 0)),
        ],
        out_specs=pl.BlockSpec((_BR, 64), lambda i: (i, 0)),
        out_shape=jax.ShapeDtypeStruct((NN, 64), jnp.float32),
    )(p5, b52, res, nd)


def kernel(x, edge_index, edge_weight, W1, b1, W2, b2, W3, b3, W4, b4,
           W5, b5, Wr, br):
    pad = NEP - NE
    src = jnp.concatenate([edge_index[0], jnp.zeros((pad,), jnp.int32)])
    dst = jnp.concatenate([edge_index[1], jnp.zeros((pad,), jnp.int32)])
    w16 = jnp.broadcast_to(
        jnp.concatenate([edge_weight, jnp.zeros((pad,), jnp.float32)])[:, None],
        (NEP, 16))

    ones = jnp.ones((NN, 128), jnp.float32)
    zeros = jnp.zeros((NNP, 128), jnp.float32)
    dego = _conv_call(ones, src, src, w16, zeros)
    degi = _conv_call(ones, dst, dst, w16, zeros)
    g1a, g1b, res, ns, nd = _t1_call(
        x, W1, Wr, br.reshape(1, 64), dego, degi)

    p1a = _conv_call(g1a, src, dst, w16, zeros)
    p1b = _conv_call(g1b, src, dst, w16, zeros)
    g2a, g2b = _tmid_call([p1a, p1b], b1.reshape(1, 256), W2, ns, nd)

    p2a = _conv_call(g2a, src, dst, w16, zeros)
    p2b = _conv_call(g2b, src, dst, w16, zeros)
    (g3,) = _tmid_call([p2a, p2b], b2.reshape(1, 256), W3, ns, nd)

    p3 = _conv_call(g3, src, dst, w16, zeros)
    (g4,) = _tmid_call([p3], b3.reshape(1, 128), W4, ns, nd)

    p4 = _conv_call(g4, src, dst, w16, zeros)
    (g5,) = _tmid_call([p4], b4.reshape(1, 128), W5, ns, nd)

    p5 = _conv_call(g5, src, dst, w16, zeros)
    return _t6_call(p5, b5.reshape(1, 64), res, nd)


# speedup vs baseline: 1.2941x; 1.1429x over previous
import jax
import jax.numpy as jnp
from jax import lax
from jax.experimental import pallas as pl
from jax.experimental.pallas import tpu as pltpu
from jax.experimental.pallas import tpu_sc as plsc

NN = 10000
NE = 160000
NC = 2
NS = 16
NWORK = NC * NS
NEP = 163840
EPW = NEP // NWORK
CHUNK = 64
NCH = EPW // CHUNK
NNP = 10240
RPS = NNP // NS

_BR = 2000


def _sc_mesh():
    return plsc.VectorSubcoreMesh(core_axis_name="c", subcore_axis_name="s")


def _make_conv_kernel():
    dh = 128
    out_t = jax.ShapeDtypeStruct((NC, NNP, dh), jnp.float32)

    @pl.kernel(out_type=out_t, mesh=_sc_mesh(),
               scratch_types=[pltpu.VMEM((CHUNK,), jnp.int32),
                              pltpu.VMEM((CHUNK,), jnp.int32),
                              pltpu.VMEM((CHUNK,), jnp.int32),
                              pltpu.VMEM((CHUNK,), jnp.int32),
                              pltpu.VMEM((CHUNK,), jnp.int32),
                              pltpu.VMEM((CHUNK,), jnp.int32),
                              pltpu.VMEM((CHUNK, 16), jnp.float32),
                              pltpu.VMEM((CHUNK, 16), jnp.float32),
                              pltpu.VMEM((CHUNK, dh), jnp.float32),
                              pltpu.VMEM((CHUNK, dh), jnp.float32),
                              pltpu.VMEM_SHARED((NNP, dh), jnp.float32),
                              pltpu.VMEM_SHARED((8,), jnp.int32),
                              pltpu.SMEM((8,), jnp.int32),
                              pltpu.SemaphoreType.DMA,
                              pltpu.SemaphoreType.DMA,
                              pltpu.SemaphoreType.DMA,
                              pltpu.SemaphoreType.DMA,
                              pltpu.SemaphoreType.DMA,
                              pltpu.SemaphoreType.DMA,
                              pltpu.SemaphoreType.DMA,
                              pltpu.SemaphoreType.DMA])
    def k(t_h, src_h, dst_h, w16_h, z_h, mode_h, out_h,
          isrc0, isrc1, idst0, idst1, idst2, idst3,
          wv0, wv1, rows0, rows1, acc, mode_v, mode_s,
          sl0, sl1, sl2, sl3, sg0, sg1, ss0, ss1):
        c = lax.axis_index("c")
        s = lax.axis_index("s")
        wid = c * NS + s
        isrc = (isrc0, isrc1)
        idst = (idst0, idst1, idst2, idst3)
        wv = (wv0, wv1)
        rows = (rows0, rows1)
        sem_l = (sl0, sl1, sl2, sl3)
        sem_g = (sg0, sg1)
        sem_s = (ss0, ss1)

        def start_loads(u, kk):
            b, i = u % 2, u % 4
            base = wid * EPW + kk * CHUNK
            pltpu.async_copy(src_h.at[pl.ds(base, CHUNK)], isrc[b], sem_l[i])
            pltpu.async_copy(dst_h.at[pl.ds(base, CHUNK)], idst[i], sem_l[i])
            pltpu.async_copy(w16_h.at[pl.ds(base, CHUNK)], wv[b], sem_l[i])

        def wait_loads(u):
            b, i = u % 2, u % 4
            pltpu.make_async_copy(src_h.at[pl.ds(0, CHUNK)], isrc[b],
                                  sem_l[i]).wait()
            pltpu.make_async_copy(dst_h.at[pl.ds(0, CHUNK)], idst[i],
                                  sem_l[i]).wait()
            pltpu.make_async_copy(w16_h.at[pl.ds(0, CHUNK)], wv[b],
                                  sem_l[i]).wait()

        def multiply(b, i):
            rv, wvb = rows[b], wv[b]

            @pl.loop(0, CHUNK, step=4)
            def _(ii):
                ws = [wvb[ii + e, :] for e in range(4)]
                for j in range(dh // 16):
                    sl = pl.ds(j * 16, 16)
                    for e in range(4):
                        rv[ii + e, sl] = rv[ii + e, sl] * ws[e]

        def fill(b):
            rv, wvb = rows[b], wv[b]

            @pl.loop(0, CHUNK, step=4)
            def _(ii):
                for e in range(4):
                    rv[ii + e, pl.ds(0, 16)] = wvb[ii + e, :]

        def start_gather(b, i):
            pltpu.async_copy(t_h.at[isrc[b]], rows[b], sem_g[b])

        def wait_gather(b, i):
            pltpu.make_async_copy(t_h.at[isrc[b]], rows[b], sem_g[b]).wait()

        def start_scatter(b, i):
            pltpu.async_copy(rows[b], acc.at[idst[i]], sem_s[b], add=True)

        def wait_scatter(b, i):
            pltpu.make_async_copy(rows[b], acc.at[idst[i]],
                                  sem_s[b]).wait()

        pltpu.sync_copy(mode_h, mode_v)
        pltpu.sync_copy(mode_v, mode_s)
        is_conv = mode_s[0] == 0
        pltpu.sync_copy(z_h.at[pl.ds(s * RPS, RPS)],
                        acc.at[pl.ds(s * RPS, RPS)])
        plsc.subcore_barrier()

        start_loads(0, 0)
        start_loads(1, 1)
        wait_loads(0)

        @pl.when(is_conv)
        def _():
            start_gather(0, 0)

        @pl.loop(0, NCH, step=4)
        def _(kk):
            for u in range(4):
                b = u % 2
                i = u % 4
                cc = kk + u

                @pl.when(is_conv)
                def _():
                    wait_gather(b, i)

                @pl.when(cc + 1 < NCH)
                def _():
                    wait_loads(u + 1)

                @pl.when(cc >= 1)
                def _():
                    wait_scatter(1 - b, (u + 3) % 4)

                @pl.when((cc + 1 < NCH) & is_conv)
                def _():
                    start_gather(1 - b, (u + 1) % 4)

                @pl.when(is_conv)
                def _():
                    multiply(b, i)

                @pl.when(jnp.logical_not(is_conv))
                def _():
                    fill(b)

                start_scatter(b, i)

                @pl.when(cc + 2 < NCH)
                def _():
                    start_loads(u + 2, cc + 2)

        wait_scatter(1, 3)
        plsc.subcore_barrier()
        pltpu.sync_copy(acc.at[pl.ds(s * RPS, RPS)],
                        out_h.at[c, pl.ds(s * RPS, RPS)])

    return k


_CONV_KERNEL = _make_conv_kernel()


def _conv_call(table, src, dst, w16, zeros, mode):
    return _CONV_KERNEL(table, src, dst, w16, zeros, mode)


def _norms(do_ref, di_ref):
    deg_o = do_ref[0, :, 0:1] + do_ref[1, :, 0:1]
    deg_i = di_ref[0, :, 0:1] + di_ref[1, :, 0:1]
    ns = jnp.where(deg_o > 0, lax.rsqrt(jnp.maximum(deg_o, 1e-12)), 0.0)
    nd = jnp.where(deg_i > 0, lax.rsqrt(jnp.maximum(deg_i, 1e-12)), 0.0)
    return ns, nd


def _dot(a, b):
    return jnp.dot(a, b, preferred_element_type=jnp.float32,
                   precision=lax.Precision.HIGHEST)


def _t1_call(x, W1, Wr, br2, dego, degi):
    def body(x_ref, w1_ref, wr_ref, br_ref, do_ref, di_ref,
             g1a_ref, g1b_ref, res_ref, ns_ref, nd_ref):
        ns, nd = _norms(do_ref, di_ref)
        xb = x_ref[...]
        g = ns * _dot(xb, w1_ref[...])
        g1a_ref[...] = g[:, :128]
        g1b_ref[...] = g[:, 128:]
        res_ref[...] = _dot(xb, wr_ref[...]) + br_ref[...]
        ns_ref[...] = ns
        nd_ref[...] = nd

    grid = NN // _BR
    return pl.pallas_call(
        body,
        grid=(grid,),
        in_specs=[
            pl.BlockSpec((_BR, 256), lambda i: (i, 0)),
            pl.BlockSpec((256, 256), lambda i: (0, 0)),
            pl.BlockSpec((256, 64), lambda i: (0, 0)),
            pl.BlockSpec((1, 64), lambda i: (0, 0)),
            pl.BlockSpec((NC, _BR, 128), lambda i: (0, i, 0)),
            pl.BlockSpec((NC, _BR, 128), lambda i: (0, i, 0)),
        ],
        out_specs=[
            pl.BlockSpec((_BR, 128), lambda i: (i, 0)),
            pl.BlockSpec((_BR, 128), lambda i: (i, 0)),
            pl.BlockSpec((_BR, 64), lambda i: (i, 0)),
            pl.BlockSpec((_BR, 1), lambda i: (i, 0)),
            pl.BlockSpec((_BR, 1), lambda i: (i, 0)),
        ],
        out_shape=[
            jax.ShapeDtypeStruct((NN, 128), jnp.float32),
            jax.ShapeDtypeStruct((NN, 128), jnp.float32),
            jax.ShapeDtypeStruct((NN, 64), jnp.float32),
            jax.ShapeDtypeStruct((NN, 1), jnp.float32),
            jax.ShapeDtypeStruct((NN, 1), jnp.float32),
        ],
    )(x, W1, Wr, br2, dego, degi)


def _tmid_call(parts, b2, W, ns, nd, relu=True):
    nparts = len(parts)
    dprev = sum(p.shape[2] for p in parts)
    dout = W.shape[1]
    nouts = max(1, dout // 128)

    def body(*refs):
        p_refs = refs[:nparts]
        b_ref, w_ref, ns_ref, nd_ref = refs[nparts:nparts + 4]
        o_refs = refs[nparts + 4:]
        ns_v = ns_ref[...]
        nd_v = nd_ref[...]
        g = None
        col = 0
        for kk, p_ref in enumerate(p_refs):
            dh = p_ref.shape[2]
            h = nd_v * (p_ref[0] + p_ref[1]) + b_ref[:, col:col + dh]
            if relu:
                h = jnp.maximum(h, 0.0)
            contrib = _dot(h, w_ref[col:col + dh, :])
            g = contrib if g is None else g + contrib
            col += dh
        g = ns_v * g
        if dout < 128:
            g = jnp.concatenate([g, jnp.zeros((g.shape[0], 128 - dout),
                                              jnp.float32)], axis=1)
        if nouts == 1:
            o_refs[0][...] = g
        else:
            for kk in range(nouts):
                o_refs[kk][...] = g[:, kk * 128:(kk + 1) * 128]

    grid = NN // _BR
    in_specs = [pl.BlockSpec((NC, _BR, p.shape[2]), lambda i: (0, i, 0))
                for p in parts]
    in_specs += [
        pl.BlockSpec((1, dprev), lambda i: (0, 0)),
        pl.BlockSpec((dprev, dout), lambda i: (0, 0)),
        pl.BlockSpec((_BR, 1), lambda i: (i, 0)),
        pl.BlockSpec((_BR, 1), lambda i: (i, 0)),
    ]
    ow = 128
    out_specs = [pl.BlockSpec((_BR, ow), lambda i: (i, 0))] * nouts
    out_shape = [jax.ShapeDtypeStruct((NN, ow), jnp.float32)] * nouts
    res = pl.pallas_call(
        body, grid=(grid,), in_specs=in_specs, out_specs=out_specs,
        out_shape=out_shape,
    )(*parts, b2, W, ns, nd)
    return list(res)


def _t6_call(p5, b52, res, nd):
    def body(p_ref, b_ref, r_ref, nd_ref, o_ref):
        o_ref[...] = (nd_ref[...] * (p_ref[0][:, :64] + p_ref[1][:, :64])
                      + b_ref[...] + r_ref[...])

    grid = NN // _BR
    return pl.pallas_call(
        body,
        grid=(grid,),
        in_specs=[
            pl.BlockSpec((NC, _BR, 128), lambda i: (0, i, 0)),
            pl.BlockSpec((1, 64), lambda i: (0, 0)),
            pl.BlockSpec((_BR, 64), lambda i: (i, 0)),
            pl.BlockSpec((_BR, 1), lambda i: (i, 0)),
        ],
        out_specs=pl.BlockSpec((_BR, 64), lambda i: (i, 0)),
        out_shape=jax.ShapeDtypeStruct((NN, 64), jnp.float32),
    )(p5, b52, res, nd)


def kernel(x, edge_index, edge_weight, W1, b1, W2, b2, W3, b3, W4, b4,
           W5, b5, Wr, br):
    pad = NEP - NE
    src = jnp.concatenate([edge_index[0], jnp.zeros((pad,), jnp.int32)])
    dst = jnp.concatenate([edge_index[1], jnp.zeros((pad,), jnp.int32)])
    w16 = jnp.broadcast_to(
        jnp.concatenate([edge_weight, jnp.zeros((pad,), jnp.float32)])[:, None],
        (NEP, 16))

    ones = jnp.ones((NN, 128), jnp.float32)
    zeros = jnp.zeros((NNP, 128), jnp.float32)
    m_conv = jnp.zeros((8,), jnp.int32)
    m_deg = jnp.ones((8,), jnp.int32)
    dego = _conv_call(ones, src, src, w16, zeros, m_deg)
    degi = _conv_call(ones, dst, dst, w16, zeros, m_deg)
    g1a, g1b, res, ns, nd = _t1_call(
        x, W1, Wr, br.reshape(1, 64), dego, degi)

    p1a = _conv_call(g1a, src, dst, w16, zeros, m_conv)
    p1b = _conv_call(g1b, src, dst, w16, zeros, m_conv)
    g2a, g2b = _tmid_call([p1a, p1b], b1.reshape(1, 256), W2, ns, nd)

    p2a = _conv_call(g2a, src, dst, w16, zeros, m_conv)
    p2b = _conv_call(g2b, src, dst, w16, zeros, m_conv)
    (g3,) = _tmid_call([p2a, p2b], b2.reshape(1, 256), W3, ns, nd)

    p3 = _conv_call(g3, src, dst, w16, zeros, m_conv)
    (g4,) = _tmid_call([p3], b3.reshape(1, 128), W4, ns, nd)

    p4 = _conv_call(g4, src, dst, w16, zeros, m_conv)
    (g5,) = _tmid_call([p4], b4.reshape(1, 128), W5, ns, nd)

    p5 = _conv_call(g5, src, dst, w16, zeros, m_conv)
    return _t6_call(p5, b5.reshape(1, 64), res, nd)


# speedup vs baseline: 1.3146x; 1.0159x over previous
import jax
import jax.numpy as jnp
from jax import lax
from jax.experimental import pallas as pl
from jax.experimental.pallas import tpu as pltpu
from jax.experimental.pallas import tpu_sc as plsc

NN = 10000
NE = 160000
NC = 2
NS = 16
NWORK = NC * NS
NEP = 163840
EPW = NEP // NWORK
CHUNK = 80
NCH = EPW // CHUNK
NNP = 10240
RPS = NNP // NS

_BR = 2000


def _sc_mesh():
    return plsc.VectorSubcoreMesh(core_axis_name="c", subcore_axis_name="s")


def _make_conv_kernel():
    dh = 128
    out_t = jax.ShapeDtypeStruct((NC, NNP, dh), jnp.float32)

    @pl.kernel(out_type=out_t, mesh=_sc_mesh(),
               scratch_types=[pltpu.VMEM((CHUNK,), jnp.int32),
                              pltpu.VMEM((CHUNK,), jnp.int32),
                              pltpu.VMEM((CHUNK,), jnp.int32),
                              pltpu.VMEM((CHUNK,), jnp.int32),
                              pltpu.VMEM((CHUNK,), jnp.int32),
                              pltpu.VMEM((CHUNK,), jnp.int32),
                              pltpu.VMEM((CHUNK, 16), jnp.float32),
                              pltpu.VMEM((CHUNK, 16), jnp.float32),
                              pltpu.VMEM((CHUNK, dh), jnp.float32),
                              pltpu.VMEM((CHUNK, dh), jnp.float32),
                              pltpu.VMEM_SHARED((NNP, dh), jnp.float32),
                              pltpu.VMEM_SHARED((8,), jnp.int32),
                              pltpu.SMEM((8,), jnp.int32),
                              pltpu.SemaphoreType.DMA,
                              pltpu.SemaphoreType.DMA,
                              pltpu.SemaphoreType.DMA,
                              pltpu.SemaphoreType.DMA,
                              pltpu.SemaphoreType.DMA,
                              pltpu.SemaphoreType.DMA,
                              pltpu.SemaphoreType.DMA,
                              pltpu.SemaphoreType.DMA])
    def k(t_h, src_h, dst_h, w16_h, z_h, mode_h, out_h,
          isrc0, isrc1, idst0, idst1, idst2, idst3,
          wv0, wv1, rows0, rows1, acc, mode_v, mode_s,
          sl0, sl1, sl2, sl3, sg0, sg1, ss0, ss1):
        c = lax.axis_index("c")
        s = lax.axis_index("s")
        wid = c * NS + s
        isrc = (isrc0, isrc1)
        idst = (idst0, idst1, idst2, idst3)
        wv = (wv0, wv1)
        rows = (rows0, rows1)
        sem_l = (sl0, sl1, sl2, sl3)
        sem_g = (sg0, sg1)
        sem_s = (ss0, ss1)

        def start_loads(u, kk):
            b, i = u % 2, u % 4
            base = wid * EPW + kk * CHUNK
            pltpu.async_copy(src_h.at[pl.ds(base, CHUNK)], isrc[b], sem_l[i])
            pltpu.async_copy(dst_h.at[pl.ds(base, CHUNK)], idst[i], sem_l[i])
            pltpu.async_copy(w16_h.at[pl.ds(base, CHUNK)], wv[b], sem_l[i])

        def wait_loads(u):
            b, i = u % 2, u % 4
            pltpu.make_async_copy(src_h.at[pl.ds(0, CHUNK)], isrc[b],
                                  sem_l[i]).wait()
            pltpu.make_async_copy(dst_h.at[pl.ds(0, CHUNK)], idst[i],
                                  sem_l[i]).wait()
            pltpu.make_async_copy(w16_h.at[pl.ds(0, CHUNK)], wv[b],
                                  sem_l[i]).wait()

        def multiply(b, i):
            rv, wvb = rows[b], wv[b]

            @pl.loop(0, CHUNK, step=4)
            def _(ii):
                ws = [wvb[ii + e, :] for e in range(4)]
                for j in range(dh // 16):
                    sl = pl.ds(j * 16, 16)
                    for e in range(4):
                        rv[ii + e, sl] = rv[ii + e, sl] * ws[e]

        def fill(b):
            rv, wvb = rows[b], wv[b]

            @pl.loop(0, CHUNK, step=4)
            def _(ii):
                for e in range(4):
                    rv[ii + e, pl.ds(0, 16)] = wvb[ii + e, :]

        def start_gather(b, i):
            pltpu.async_copy(t_h.at[isrc[b]], rows[b], sem_g[b])

        def wait_gather(b, i):
            pltpu.make_async_copy(t_h.at[isrc[b]], rows[b], sem_g[b]).wait()

        def start_scatter(b, i):
            pltpu.async_copy(rows[b], acc.at[idst[i]], sem_s[b], add=True)

        def wait_scatter(b, i):
            pltpu.make_async_copy(rows[b], acc.at[idst[i]],
                                  sem_s[b]).wait()

        pltpu.sync_copy(mode_h, mode_v)
        pltpu.sync_copy(mode_v, mode_s)
        is_conv = mode_s[0] == 0
        pltpu.sync_copy(z_h.at[pl.ds(s * RPS, RPS)],
                        acc.at[pl.ds(s * RPS, RPS)])
        plsc.subcore_barrier()

        start_loads(0, 0)
        start_loads(1, 1)
        wait_loads(0)

        @pl.when(is_conv)
        def _():
            start_gather(0, 0)

        @pl.loop(0, NCH, step=4)
        def _(kk):
            for u in range(4):
                b = u % 2
                i = u % 4
                cc = kk + u

                @pl.when(is_conv)
                def _():
                    wait_gather(b, i)

                @pl.when(cc + 1 < NCH)
                def _():
                    wait_loads(u + 1)

                @pl.when(cc >= 1)
                def _():
                    wait_scatter(1 - b, (u + 3) % 4)

                @pl.when((cc + 1 < NCH) & is_conv)
                def _():
                    start_gather(1 - b, (u + 1) % 4)

                @pl.when(is_conv)
                def _():
                    multiply(b, i)

                @pl.when(jnp.logical_not(is_conv))
                def _():
                    fill(b)

                start_scatter(b, i)

                @pl.when(cc + 2 < NCH)
                def _():
                    start_loads(u + 2, cc + 2)

        wait_scatter(1, 3)
        plsc.subcore_barrier()
        pltpu.sync_copy(acc.at[pl.ds(s * RPS, RPS)],
                        out_h.at[c, pl.ds(s * RPS, RPS)])

    return k


_CONV_KERNEL = _make_conv_kernel()


def _conv_call(table, src, dst, w16, zeros, mode):
    return _CONV_KERNEL(table, src, dst, w16, zeros, mode)


def _norms(do_ref, di_ref):
    deg_o = do_ref[0, :, 0:1] + do_ref[1, :, 0:1]
    deg_i = di_ref[0, :, 0:1] + di_ref[1, :, 0:1]
    ns = jnp.where(deg_o > 0, lax.rsqrt(jnp.maximum(deg_o, 1e-12)), 0.0)
    nd = jnp.where(deg_i > 0, lax.rsqrt(jnp.maximum(deg_i, 1e-12)), 0.0)
    return ns, nd


def _dot(a, b):
    return jnp.dot(a, b, preferred_element_type=jnp.float32,
                   precision=lax.Precision.HIGHEST)


def _t1_call(x, W1, Wr, br2, dego, degi):
    def body(x_ref, w1_ref, wr_ref, br_ref, do_ref, di_ref,
             g1a_ref, g1b_ref, res_ref, ns_ref, nd_ref):
        ns, nd = _norms(do_ref, di_ref)
        xb = x_ref[...]
        g = ns * _dot(xb, w1_ref[...])
        g1a_ref[...] = g[:, :128]
        g1b_ref[...] = g[:, 128:]
        res_ref[...] = _dot(xb, wr_ref[...]) + br_ref[...]
        ns_ref[...] = ns
        nd_ref[...] = nd

    grid = NN // _BR
    return pl.pallas_call(
        body,
        grid=(grid,),
        in_specs=[
            pl.BlockSpec((_BR, 256), lambda i: (i, 0)),
            pl.BlockSpec((256, 256), lambda i: (0, 0)),
            pl.BlockSpec((256, 64), lambda i: (0, 0)),
            pl.BlockSpec((1, 64), lambda i: (0, 0)),
            pl.BlockSpec((NC, _BR, 128), lambda i: (0, i, 0)),
            pl.BlockSpec((NC, _BR, 128), lambda i: (0, i, 0)),
        ],
        out_specs=[
            pl.BlockSpec((_BR, 128), lambda i: (i, 0)),
            pl.BlockSpec((_BR, 128), lambda i: (i, 0)),
            pl.BlockSpec((_BR, 64), lambda i: (i, 0)),
            pl.BlockSpec((_BR, 1), lambda i: (i, 0)),
            pl.BlockSpec((_BR, 1), lambda i: (i, 0)),
        ],
        out_shape=[
            jax.ShapeDtypeStruct((NN, 128), jnp.float32),
            jax.ShapeDtypeStruct((NN, 128), jnp.float32),
            jax.ShapeDtypeStruct((NN, 64), jnp.float32),
            jax.ShapeDtypeStruct((NN, 1), jnp.float32),
            jax.ShapeDtypeStruct((NN, 1), jnp.float32),
        ],
    )(x, W1, Wr, br2, dego, degi)


def _tmid_call(parts, b2, W, ns, nd, relu=True):
    nparts = len(parts)
    dprev = sum(p.shape[2] for p in parts)
    dout = W.shape[1]
    nouts = max(1, dout // 128)

    def body(*refs):
        p_refs = refs[:nparts]
        b_ref, w_ref, ns_ref, nd_ref = refs[nparts:nparts + 4]
        o_refs = refs[nparts + 4:]
        ns_v = ns_ref[...]
        nd_v = nd_ref[...]
        g = None
        col = 0
        for kk, p_ref in enumerate(p_refs):
            dh = p_ref.shape[2]
            h = nd_v * (p_ref[0] + p_ref[1]) + b_ref[:, col:col + dh]
            if relu:
                h = jnp.maximum(h, 0.0)
            contrib = _dot(h, w_ref[col:col + dh, :])
            g = contrib if g is None else g + contrib
            col += dh
        g = ns_v * g
        if dout < 128:
            g = jnp.concatenate([g, jnp.zeros((g.shape[0], 128 - dout),
                                              jnp.float32)], axis=1)
        if nouts == 1:
            o_refs[0][...] = g
        else:
            for kk in range(nouts):
                o_refs[kk][...] = g[:, kk * 128:(kk + 1) * 128]

    grid = NN // _BR
    in_specs = [pl.BlockSpec((NC, _BR, p.shape[2]), lambda i: (0, i, 0))
                for p in parts]
    in_specs += [
        pl.BlockSpec((1, dprev), lambda i: (0, 0)),
        pl.BlockSpec((dprev, dout), lambda i: (0, 0)),
        pl.BlockSpec((_BR, 1), lambda i: (i, 0)),
        pl.BlockSpec((_BR, 1), lambda i: (i, 0)),
    ]
    ow = 128
    out_specs = [pl.BlockSpec((_BR, ow), lambda i: (i, 0))] * nouts
    out_shape = [jax.ShapeDtypeStruct((NN, ow), jnp.float32)] * nouts
    res = pl.pallas_call(
        body, grid=(grid,), in_specs=in_specs, out_specs=out_specs,
        out_shape=out_shape,
    )(*parts, b2, W, ns, nd)
    return list(res)


def _t6_call(p5, b52, res, nd):
    def body(p_ref, b_ref, r_ref, nd_ref, o_ref):
        o_ref[...] = (nd_ref[...] * (p_ref[0][:, :64] + p_ref[1][:, :64])
                      + b_ref[...] + r_ref[...])

    grid = NN // _BR
    return pl.pallas_call(
        body,
        grid=(grid,),
        in_specs=[
            pl.BlockSpec((NC, _BR, 128), lambda i: (0, i, 0)),
            pl.BlockSpec((1, 64), lambda i: (0, 0)),
            pl.BlockSpec((_BR, 64), lambda i: (i, 0)),
            pl.BlockSpec((_BR, 1), lambda i: (i, 0)),
        ],
        out_specs=pl.BlockSpec((_BR, 64), lambda i: (i, 0)),
        out_shape=jax.ShapeDtypeStruct((NN, 64), jnp.float32),
    )(p5, b52, res, nd)


def kernel(x, edge_index, edge_weight, W1, b1, W2, b2, W3, b3, W4, b4,
           W5, b5, Wr, br):
    pad = NEP - NE
    src = jnp.concatenate([edge_index[0], jnp.zeros((pad,), jnp.int32)])
    dst = jnp.concatenate([edge_index[1], jnp.zeros((pad,), jnp.int32)])
    w16 = jnp.broadcast_to(
        jnp.concatenate([edge_weight, jnp.zeros((pad,), jnp.float32)])[:, None],
        (NEP, 16))

    ones = jnp.ones((NN, 128), jnp.float32)
    zeros = jnp.zeros((NNP, 128), jnp.float32)
    m_conv = jnp.zeros((8,), jnp.int32)
    m_deg = jnp.ones((8,), jnp.int32)
    dego = _conv_call(ones, src, src, w16, zeros, m_deg)
    degi = _conv_call(ones, dst, dst, w16, zeros, m_deg)
    g1a, g1b, res, ns, nd = _t1_call(
        x, W1, Wr, br.reshape(1, 64), dego, degi)

    p1a = _conv_call(g1a, src, dst, w16, zeros, m_conv)
    p1b = _conv_call(g1b, src, dst, w16, zeros, m_conv)
    g2a, g2b = _tmid_call([p1a, p1b], b1.reshape(1, 256), W2, ns, nd)

    p2a = _conv_call(g2a, src, dst, w16, zeros, m_conv)
    p2b = _conv_call(g2b, src, dst, w16, zeros, m_conv)
    (g3,) = _tmid_call([p2a, p2b], b2.reshape(1, 256), W3, ns, nd)

    p3 = _conv_call(g3, src, dst, w16, zeros, m_conv)
    (g4,) = _tmid_call([p3], b3.reshape(1, 128), W4, ns, nd)

    p4 = _conv_call(g4, src, dst, w16, zeros, m_conv)
    (g5,) = _tmid_call([p4], b4.reshape(1, 128), W5, ns, nd)

    p5 = _conv_call(g5, src, dst, w16, zeros, m_conv)
    return _t6_call(p5, b5.reshape(1, 64), res, nd)
